# initial kernel scaffold (unmeasured)
import functools

import jax
import jax.numpy as jnp
from jax import lax
from jax.experimental import pallas as pl
from jax.experimental.pallas import tpu as pltpu

N_DEV = 4
M_PER = 1024
K = 4096
N_PER = 2048
NB = 512
NSUB = N_PER // NB


def _silu(y):
    return y * jax.nn.sigmoid(y)


def kernel(x, w_mat):
    def body(x_hbm, w_hbm, out_hbm, stage, x_bf, w_blk,
             send_buf, recv_buf, send_sems, recv_sems, local_sem):
        me = lax.axis_index("i")

        barrier_sem = pltpu.get_barrier_semaphore()
        for j in range(1, N_DEV):
            pl.semaphore_signal(
                barrier_sem, inc=1,
                device_id=((me + j) % N_DEV,),
                device_id_type=pl.DeviceIdType.MESH,
            )
        pl.semaphore_wait(barrier_sem, N_DEV - 1)

        for h in range(2):
            cp = pltpu.make_async_copy(
                x_hbm.at[:, pl.ds(h * N_PER, N_PER)], stage, local_sem)
            cp.start()
            cp.wait()
            x_bf[:, h * N_PER:(h + 1) * N_PER] = stage[...].astype(jnp.bfloat16)

        xv = x_bf[...]

        def compute_chunk(t, out_f32_ref, out_bf16_ref):
            for b in range(NSUB):
                col = t * N_PER + b * NB
                cp = pltpu.make_async_copy(
                    w_hbm.at[:, pl.ds(col, NB)], w_blk, local_sem)
                cp.start()
                cp.wait()
                y = jnp.dot(xv, w_blk[...].astype(jnp.bfloat16),
                            preferred_element_type=jnp.float32)
                z = _silu(y)
                if out_f32_ref is not None:
                    out_f32_ref[:, b * NB:(b + 1) * NB] = z
                else:
                    out_bf16_ref[:, b * NB:(b + 1) * NB] = z.astype(jnp.bfloat16)

        for j in range(1, N_DEV):
            t = (me + j) % N_DEV
            slot = (j - 1) % 2
            compute_chunk(t, None, send_buf.at[slot])
            rdma = pltpu.make_async_remote_copy(
                src_ref=send_buf.at[slot],
                dst_ref=recv_buf.at[j - 1],
                send_sem=send_sems.at[slot],
                recv_sem=recv_sems.at[j - 1],
                device_id=(t,),
                device_id_type=pl.DeviceIdType.MESH,
            )
            rdma.start()
            rdma.wait()

        compute_chunk(me, stage, None)
        cp = pltpu.make_async_copy(stage, out_hbm.at[me], local_sem)
        cp.start()
        cp.wait()

        for j in range(1, N_DEV):
            s = (me - j) % N_DEV
            stage[...] = recv_buf[j - 1].astype(jnp.float32)
            cp = pltpu.make_async_copy(stage, out_hbm.at[s], local_sem)
            cp.start()
            cp.wait()

    out = pl.pallas_call(
        body,
        out_shape=jax.ShapeDtypeStruct((N_DEV, M_PER, N_PER), jnp.float32),
        in_specs=[
            pl.BlockSpec(memory_space=pltpu.ANY),
            pl.BlockSpec(memory_space=pltpu.ANY),
        ],
        out_specs=pl.BlockSpec(memory_space=pltpu.ANY),
        scratch_shapes=[
            pltpu.VMEM((M_PER, N_PER), jnp.float32),
            pltpu.VMEM((M_PER, K), jnp.bfloat16),
            pltpu.VMEM((K, NB), jnp.float32),
            pltpu.VMEM((2, M_PER, N_PER), jnp.bfloat16),
            pltpu.VMEM((N_DEV - 1, M_PER, N_PER), jnp.bfloat16),
            pltpu.SemaphoreType.DMA((2,)),
            pltpu.SemaphoreType.DMA((N_DEV - 1,)),
            pltpu.SemaphoreType.DMA,
        ],
        compiler_params=pltpu.CompilerParams(collective_id=0),
    )(x, w_mat)
    return out.reshape(N_DEV * M_PER, N_PER)


# baseline (device time: 197913 ns/iter reference)
import jax
import jax.numpy as jnp
from jax import lax
from jax.experimental import pallas as pl
from jax.experimental.pallas import tpu as pltpu

N_DEV = 4
M_PER = 1024
K = 4096
N_PER = 2048
NB = 256
NSUB = N_PER // NB
SEND_COLS = 1024
NSEND = N_PER // SEND_COLS
SUB_PER_SEND = SEND_COLS // NB


def _silu(y):
    return y * jax.nn.sigmoid(y)


def kernel(x, w_mat):
    def body(x_hbm, w_hbm, out_hbm, stage, x_bf, w_blk,
             send_buf, recv_buf, send_sems, recv_sems, w_sems, local_sem):
        me = lax.axis_index("i")

        barrier_sem = pltpu.get_barrier_semaphore()
        for j in range(1, N_DEV):
            pl.semaphore_signal(
                barrier_sem, inc=1,
                device_id=((me + j) % N_DEV,),
                device_id_type=pl.DeviceIdType.MESH,
            )
        pl.semaphore_wait(barrier_sem, N_DEV - 1)

        for h in range(2):
            cp = pltpu.make_async_copy(
                x_hbm.at[:, pl.ds(h * N_PER, N_PER)], stage, local_sem)
            cp.start()
            cp.wait()
            x_bf[:, h * N_PER:(h + 1) * N_PER] = stage[...].astype(jnp.bfloat16)

        xv = x_bf[...]

        def target_of(c):
            j = c + 1
            return (me + j) % N_DEV

        def w_dma(idx):
            c, b = divmod(idx, NSUB)
            col = target_of(c) * N_PER + b * NB
            return pltpu.make_async_copy(
                w_hbm.at[:, pl.ds(col, NB)],
                w_blk.at[idx % 2], w_sems.at[idx % 2])

        def rdma_for(j, h):
            cs = slice(h * SEND_COLS, (h + 1) * SEND_COLS)
            return pltpu.make_async_remote_copy(
                src_ref=send_buf.at[j - 1, :, cs],
                dst_ref=recv_buf.at[j - 1, :, cs],
                send_sem=send_sems.at[j - 1, h],
                recv_sem=recv_sems.at[j - 1, h],
                device_id=((me + j) % N_DEV,),
                device_id_type=pl.DeviceIdType.MESH,
            )

        w_dma(0).start()
        for idx in range(N_DEV * NSUB):
            c, b = divmod(idx, NSUB)
            if idx + 1 < N_DEV * NSUB:
                w_dma(idx + 1).start()
            w_dma(idx).wait()
            y = jnp.dot(xv, w_blk[idx % 2].astype(jnp.bfloat16),
                        preferred_element_type=jnp.float32)
            z = _silu(y)
            cs = slice(b * NB, (b + 1) * NB)
            if c < N_DEV - 1:
                send_buf[c, :, cs] = z.astype(jnp.bfloat16)
                if (b + 1) % SUB_PER_SEND == 0:
                    rdma_for(c + 1, b // SUB_PER_SEND).start()
            else:
                stage[:, cs] = z

        own_cp = pltpu.make_async_copy(stage, out_hbm.at[me], local_sem)
        own_cp.start()
        own_cp.wait()

        for j in range(1, N_DEV):
            for h in range(NSEND):
                rdma_for(j, h).wait_recv()
            s = (me - j) % N_DEV
            stage[...] = recv_buf[j - 1].astype(jnp.float32)
            cp = pltpu.make_async_copy(stage, out_hbm.at[s], local_sem)
            cp.start()
            cp.wait()

        for j in range(1, N_DEV):
            for h in range(NSEND):
                rdma_for(j, h).wait_send()

    out = pl.pallas_call(
        body,
        out_shape=jax.ShapeDtypeStruct((N_DEV, M_PER, N_PER), jnp.float32),
        in_specs=[
            pl.BlockSpec(memory_space=pltpu.MemorySpace.HBM),
            pl.BlockSpec(memory_space=pltpu.MemorySpace.HBM),
        ],
        out_specs=pl.BlockSpec(memory_space=pltpu.MemorySpace.HBM),
        scratch_shapes=[
            pltpu.VMEM((M_PER, N_PER), jnp.float32),
            pltpu.VMEM((M_PER, K), jnp.bfloat16),
            pltpu.VMEM((2, K, NB), jnp.float32),
            pltpu.VMEM((N_DEV - 1, M_PER, N_PER), jnp.bfloat16),
            pltpu.VMEM((N_DEV - 1, M_PER, N_PER), jnp.bfloat16),
            pltpu.SemaphoreType.DMA((N_DEV - 1, NSEND)),
            pltpu.SemaphoreType.DMA((N_DEV - 1, NSEND)),
            pltpu.SemaphoreType.DMA((2,)),
            pltpu.SemaphoreType.DMA,
        ],
        compiler_params=pltpu.CompilerParams(
            collective_id=0, vmem_limit_bytes=64 * 1024 * 1024),
    )(x, w_mat)
    return out.reshape(N_DEV * M_PER, N_PER)


# device time: 166415 ns/iter; 1.1893x vs baseline; 1.1893x over previous
import jax
import jax.numpy as jnp
from jax import lax
from jax.experimental import pallas as pl
from jax.experimental.pallas import tpu as pltpu

N_DEV = 4
M_PER = 1024
K = 4096
KH = K // 4
N_PER = 2048
NB = 512
NSUB = N_PER // NB
NKH = K // KH
NHB = NKH * N_DEV * NSUB


def _silu(y):
    return y * jax.nn.sigmoid(y)


def kernel(x, w_mat):
    def body(x_hbm, w_hbm, out_hbm, stage, own_buf, x_bf, w_blk,
             send_buf, recv_buf, send_sems, recv_sems, w_sems, own_sems,
             local_sem):
        me = lax.axis_index("i")

        barrier_sem = pltpu.get_barrier_semaphore()
        for j in range(1, N_DEV):
            pl.semaphore_signal(
                barrier_sem, inc=1,
                device_id=((me + j) % N_DEV,),
                device_id_type=pl.DeviceIdType.MESH,
            )
        pl.semaphore_wait(barrier_sem, N_DEV - 1)

        for q in range(4):
            cp = pltpu.make_async_copy(
                x_hbm.at[:, pl.ds(q * 1024, 1024)], stage, local_sem)
            cp.start()
            cp.wait()
            x_bf[:, q * 1024:(q + 1) * 1024] = stage[...].astype(jnp.bfloat16)

        def w_dma(hidx):
            sub, kh = divmod(hidx, NKH)
            c, b = divmod(sub, NSUB)
            col = ((me + 1 + c) % N_DEV) * N_PER + b * NB
            return pltpu.make_async_copy(
                w_hbm.at[pl.ds(kh * KH, KH), pl.ds(col, NB)],
                w_blk.at[hidx % 4], w_sems.at[hidx % 4])

        def rdma_for(j, b):
            cs = slice(b * NB, (b + 1) * NB)
            return pltpu.make_async_remote_copy(
                src_ref=send_buf.at[j - 1, :, cs],
                dst_ref=recv_buf.at[j - 1, :, cs],
                send_sem=send_sems.at[j - 1, b],
                recv_sem=recv_sems.at[j - 1, b],
                device_id=((me + j) % N_DEV,),
                device_id_type=pl.DeviceIdType.MESH,
            )

        def own_cp(b):
            return pltpu.make_async_copy(
                own_buf.at[b % 2],
                out_hbm.at[me, :, pl.ds(b * NB, NB)],
                own_sems.at[b % 2])

        w_dma(0).start()
        w_dma(1).start()
        y0 = None
        for hidx in range(NHB):
            sub, kh = divmod(hidx, NKH)
            c, b = divmod(sub, NSUB)
            w_dma(hidx).wait()
            if hidx + 2 < NHB:
                w_dma(hidx + 2).start()
            xs = x_bf[:, kh * KH:(kh + 1) * KH]
            partial = jnp.dot(xs, w_blk[hidx % 4].astype(jnp.bfloat16),
                              preferred_element_type=jnp.float32)
            if kh == 0:
                y0 = partial
                continue
            if kh < NKH - 1:
                y0 = y0 + partial
                continue
            z = _silu(y0 + partial)
            if c < N_DEV - 1:
                send_buf[c, :, b * NB:(b + 1) * NB] = z.astype(jnp.bfloat16)
                if (c, b) != (0, 0):
                    pj, pb = (c, b - 1) if b > 0 else (c - 1, NSUB - 1)
                    rdma_for(pj + 1, pb).start()
            else:
                if (c, b) == (N_DEV - 1, 0):
                    rdma_for(N_DEV - 1, NSUB - 1).start()
            if c == N_DEV - 1:
                if b >= 2:
                    own_cp(b - 2).wait()
                own_buf[b % 2] = z
                own_cp(b).start()
        own_cp(NSUB - 2).wait()
        own_cp(NSUB - 1).wait()

        for j in range(1, N_DEV):
            for b in range(NSUB):
                rdma_for(j, b).wait_recv()
            s = (me - j) % N_DEV
            for h in range(2):
                stage[...] = recv_buf[
                    j - 1, :, h * 1024:(h + 1) * 1024].astype(jnp.float32)
                cp = pltpu.make_async_copy(
                    stage, out_hbm.at[s, :, pl.ds(h * 1024, 1024)], local_sem)
                cp.start()
                cp.wait()

        for j in range(1, N_DEV):
            for b in range(NSUB):
                rdma_for(j, b).wait_send()

    out = pl.pallas_call(
        body,
        out_shape=jax.ShapeDtypeStruct((N_DEV, M_PER, N_PER), jnp.float32),
        in_specs=[
            pl.BlockSpec(memory_space=pltpu.MemorySpace.HBM),
            pl.BlockSpec(memory_space=pltpu.MemorySpace.HBM),
        ],
        out_specs=pl.BlockSpec(memory_space=pltpu.MemorySpace.HBM),
        scratch_shapes=[
            pltpu.VMEM((M_PER, 1024), jnp.float32),
            pltpu.VMEM((2, M_PER, NB), jnp.float32),
            pltpu.VMEM((M_PER, K), jnp.bfloat16),
            pltpu.VMEM((4, KH, NB), jnp.float32),
            pltpu.VMEM((N_DEV - 1, M_PER, N_PER), jnp.bfloat16),
            pltpu.VMEM((N_DEV - 1, M_PER, N_PER), jnp.bfloat16),
            pltpu.SemaphoreType.DMA((N_DEV - 1, NSUB)),
            pltpu.SemaphoreType.DMA((N_DEV - 1, NSUB)),
            pltpu.SemaphoreType.DMA((4,)),
            pltpu.SemaphoreType.DMA((2,)),
            pltpu.SemaphoreType.DMA,
        ],
        compiler_params=pltpu.CompilerParams(
            collective_id=0, vmem_limit_bytes=64 * 1024 * 1024),
    )(x, w_mat)
    return out.reshape(N_DEV * M_PER, N_PER)


# device time: 155445 ns/iter; 1.2732x vs baseline; 1.0706x over previous
import jax
import jax.numpy as jnp
from jax import lax
from jax.experimental import pallas as pl
from jax.experimental.pallas import tpu as pltpu

N_DEV = 4
M_PER = 1024
K = 4096
KH = K // 4
N_PER = 2048
NB = 512
NSUB = N_PER // NB
NKH = K // KH
NHB = NKH * N_DEV * NSUB


def _silu(y):
    return y * jax.nn.sigmoid(y)


def kernel(x, w_mat):
    def body(x_hbm, w_hbm, out_hbm, stage, own_buf, x_bf, w_blk,
             send_buf, recv_buf, send_sems, recv_sems, w_sems, own_sems,
             stage_sems):
        me = lax.axis_index("i")

        def w_dma(hidx):
            sub, kh = divmod(hidx, NKH)
            c, b = divmod(sub, NSUB)
            col = ((me + 1 + c) % N_DEV) * N_PER + b * NB
            return pltpu.make_async_copy(
                w_hbm.at[pl.ds(kh * KH, KH), pl.ds(col, NB)],
                w_blk.at[hidx % 4], w_sems.at[hidx % 4])

        def rdma_for(j, b):
            cs = slice(b * NB, (b + 1) * NB)
            return pltpu.make_async_remote_copy(
                src_ref=send_buf.at[j - 1, :, cs],
                dst_ref=recv_buf.at[j - 1, :, cs],
                send_sem=send_sems.at[j - 1, b],
                recv_sem=recv_sems.at[j - 1, b],
                device_id=((me + j) % N_DEV,),
                device_id_type=pl.DeviceIdType.MESH,
            )

        def own_cp(b):
            return pltpu.make_async_copy(
                own_buf.at[b % 2],
                out_hbm.at[me, :, pl.ds(b * NB, NB)],
                own_sems.at[b % 2])

        def x_dma(q):
            return pltpu.make_async_copy(
                x_hbm.at[:, pl.ds(q * 1024, 1024)],
                stage.at[q % 2], stage_sems.at[q % 2])

        w_dma(0).start()
        w_dma(1).start()
        x_dma(0).start()
        x_dma(1).start()

        barrier_sem = pltpu.get_barrier_semaphore()
        for j in range(1, N_DEV):
            pl.semaphore_signal(
                barrier_sem, inc=1,
                device_id=((me + j) % N_DEV,),
                device_id_type=pl.DeviceIdType.MESH,
            )
        pl.semaphore_wait(barrier_sem, N_DEV - 1)

        for q in range(4):
            x_dma(q).wait()
            x_bf[:, q * 1024:(q + 1) * 1024] = (
                stage[q % 2].astype(jnp.bfloat16))
            if q + 2 < 4:
                x_dma(q + 2).start()

        y0 = None
        for hidx in range(NHB):
            sub, kh = divmod(hidx, NKH)
            c, b = divmod(sub, NSUB)
            w_dma(hidx).wait()
            if hidx + 2 < NHB:
                w_dma(hidx + 2).start()
            xs = x_bf[:, kh * KH:(kh + 1) * KH]
            partial = jnp.dot(xs, w_blk[hidx % 4].astype(jnp.bfloat16),
                              preferred_element_type=jnp.float32)
            if kh == 0:
                y0 = partial
                continue
            if kh < NKH - 1:
                y0 = y0 + partial
                continue
            z = _silu(y0 + partial)
            if c < N_DEV - 1:
                send_buf[c, :, b * NB:(b + 1) * NB] = z.astype(jnp.bfloat16)
                rdma_for(c + 1, b).start()
            if c == N_DEV - 1:
                if b >= 2:
                    own_cp(b - 2).wait()
                own_buf[b % 2] = z
                own_cp(b).start()
        own_cp(NSUB - 2).wait()
        own_cp(NSUB - 1).wait()

        def drain_cp(j, h, s):
            g = 2 * (j - 1) + h
            return pltpu.make_async_copy(
                stage.at[g % 2],
                out_hbm.at[s, :, pl.ds(h * 1024, 1024)], stage_sems.at[g % 2])

        for j in range(1, N_DEV):
            for b in range(NSUB):
                rdma_for(j, b).wait_recv()
            s = (me - j) % N_DEV
            for h in range(2):
                g = 2 * (j - 1) + h
                if g >= 2:
                    pg = g - 2
                    drain_cp(1 + pg // 2, pg % 2, (me - (1 + pg // 2)) % N_DEV).wait()
                stage[g % 2] = recv_buf[
                    j - 1, :, h * 1024:(h + 1) * 1024].astype(jnp.float32)
                drain_cp(j, h, s).start()
        drain_cp(3, 0, (me - 3) % N_DEV).wait()
        drain_cp(3, 1, (me - 3) % N_DEV).wait()

        for j in range(1, N_DEV):
            for b in range(NSUB):
                rdma_for(j, b).wait_send()

    out = pl.pallas_call(
        body,
        out_shape=jax.ShapeDtypeStruct((N_DEV, M_PER, N_PER), jnp.float32),
        in_specs=[
            pl.BlockSpec(memory_space=pltpu.MemorySpace.HBM),
            pl.BlockSpec(memory_space=pltpu.MemorySpace.HBM),
        ],
        out_specs=pl.BlockSpec(memory_space=pltpu.MemorySpace.HBM),
        scratch_shapes=[
            pltpu.VMEM((2, M_PER, 1024), jnp.float32),
            pltpu.VMEM((2, M_PER, NB), jnp.float32),
            pltpu.VMEM((M_PER, K), jnp.bfloat16),
            pltpu.VMEM((4, KH, NB), jnp.float32),
            pltpu.VMEM((N_DEV - 1, M_PER, N_PER), jnp.bfloat16),
            pltpu.VMEM((N_DEV - 1, M_PER, N_PER), jnp.bfloat16),
            pltpu.SemaphoreType.DMA((N_DEV - 1, NSUB)),
            pltpu.SemaphoreType.DMA((N_DEV - 1, NSUB)),
            pltpu.SemaphoreType.DMA((4,)),
            pltpu.SemaphoreType.DMA((2,)),
            pltpu.SemaphoreType.DMA((2,)),
        ],
        compiler_params=pltpu.CompilerParams(
            collective_id=0, vmem_limit_bytes=64 * 1024 * 1024),
    )(x, w_mat)
    return out.reshape(N_DEV * M_PER, N_PER)
